# baseline (speedup 1.0000x reference)
import jax
import jax.numpy as jnp
from jax import lax
from jax.experimental import pallas as pl
from jax.experimental.pallas import tpu as pltpu
from jax.experimental.pallas import tpu_sc as plsc

BATCH = 4096
VOCAB = 100000
SEQ = 50
DIM = 64
NC = 2
NS = 16
NW = NC * NS
BPW = BATCH // NW
NBUF = 4
LANES = 16
SEQP = 56


def _sc_body(idx_hbm, table_hbm, out_hbm, idx_v, rows_v, out_v, *sems):
    wid = lax.axis_index("s") * NC + lax.axis_index("c")
    pltpu.sync_copy(idx_hbm.at[pl.ds(wid * (BPW * SEQP), BPW * SEQP)], idx_v)

    def issue(r, b):
        pltpu.async_copy(table_hbm.at[idx_v.at[pl.ds(r * SEQP, SEQ)]],
                         rows_v.at[b], sems[b])

    def consume(r, b):
        pltpu.make_async_copy(table_hbm.at[idx_v.at[pl.ds(r * SEQP, SEQ)]],
                              rows_v.at[b], sems[b]).wait()
        rb = rows_v.at[b]
        for c2 in range(2):
            col = pl.ds(c2 * 32, 32)
            a0, b0 = plsc.unpack(rb[0, col],
                                 format=plsc.PackFormat.INTERLEAVED,
                                 preferred_element_type=jnp.float32)
            a1, b1 = plsc.unpack(rb[1, col],
                                 format=plsc.PackFormat.INTERLEAVED,
                                 preferred_element_type=jnp.float32)
            for k in range(2, SEQ, 2):
                ae, be = plsc.unpack(rb[k, col],
                                     format=plsc.PackFormat.INTERLEAVED,
                                     preferred_element_type=jnp.float32)
                ao, bo = plsc.unpack(rb[k + 1, col],
                                     format=plsc.PackFormat.INTERLEAVED,
                                     preferred_element_type=jnp.float32)
                a0 += ae
                b0 += be
                a1 += ao
                b1 += bo
            base = r * DIM + c2 * 32
            out_v[pl.ds(base, LANES)] = (a0 + a1) * (1.0 / SEQ)
            out_v[pl.ds(base + LANES, LANES)] = (b0 + b1) * (1.0 / SEQ)

    for b in range(NBUF):
        issue(b, b)

    groups = BPW // NBUF

    def group(g, issue_next):
        for b in range(NBUF):
            r = g * NBUF + b
            consume(r, b)
            if issue_next:
                issue(r + NBUF, b)

    def steady(g, carry):
        group(g, True)
        return carry

    lax.fori_loop(0, groups - 1, steady, 0)
    group(groups - 1, False)

    pltpu.sync_copy(out_v, out_hbm.at[pl.ds(wid * (BPW * DIM), BPW * DIM)])


_run = pl.kernel(
    _sc_body,
    out_type=jax.ShapeDtypeStruct((BATCH * DIM,), jnp.float32),
    mesh=plsc.VectorSubcoreMesh(core_axis_name="c", subcore_axis_name="s",
                                num_cores=NC, num_subcores=NS),
    scratch_types=[
        pltpu.VMEM((BPW * SEQP,), jnp.int32),
        pltpu.VMEM((NBUF, SEQ, DIM), jnp.bfloat16),
        pltpu.VMEM((BPW * DIM,), jnp.float32),
    ] + [pltpu.SemaphoreType.DMA] * NBUF,
    compiler_params=pltpu.CompilerParams(use_tc_tiling_on_sc=False,
                                          needs_layout_passes=False),
)


def kernel(input_seq_batch, table):
    idx = jnp.pad(input_seq_batch.astype(jnp.int32),
                  ((0, 0), (0, SEQP - SEQ))).reshape(BATCH * SEQP)
    tbl = (table.astype(jnp.bfloat16)
           .reshape(VOCAB, 2, 2, LANES)
           .transpose(0, 1, 3, 2)
           .reshape(VOCAB, DIM))
    return _run(idx, tbl).reshape(BATCH, DIM)


# speedup vs baseline: 1.4777x; 1.4777x over previous
import jax
import jax.numpy as jnp
from jax import lax
from jax.experimental import pallas as pl
from jax.experimental.pallas import tpu as pltpu
from jax.experimental.pallas import tpu_sc as plsc

BATCH = 4096
VOCAB = 100000
SEQ = 50
DIM = 64
NC = 2
NS = 16
NW = NC * NS
BPW = BATCH // NW
NBUF = 8
LANES = 16
SEQP = 56


def _sc_body(idx_hbm, table_hbm, out_hbm, idx_v, rows_v, out_v, *sems):
    wid = lax.axis_index("s") * NC + lax.axis_index("c")
    pltpu.sync_copy(idx_hbm.at[pl.ds(wid * (BPW * SEQP), BPW * SEQP)], idx_v)

    def issue(r, b):
        pltpu.async_copy(table_hbm.at[idx_v.at[pl.ds(r * SEQP, SEQ)]],
                         rows_v.at[b], sems[b])

    def consume(r, b):
        pltpu.make_async_copy(table_hbm.at[idx_v.at[pl.ds(r * SEQP, SEQ)]],
                              rows_v.at[b], sems[b]).wait()
        rb = rows_v.at[b]
        for c in range(DIM // LANES):
            col = pl.ds(c * LANES, LANES)
            s0 = rb[0, col]
            s1 = rb[1, col]
            for k in range(2, SEQ, 2):
                s0 += rb[k, col]
                s1 += rb[k + 1, col]
            out_v[pl.ds(r * DIM + c * LANES, LANES)] = (s0 + s1) * (1.0 / SEQ)

    for b in range(NBUF):
        issue(b, b)

    groups = BPW // NBUF

    def group(g, issue_next):
        for b in range(NBUF):
            r = g * NBUF + b
            consume(r, b)
            if issue_next:
                issue(r + NBUF, b)

    def steady(g, carry):
        group(g, True)
        return carry

    lax.fori_loop(0, groups - 1, steady, 0)
    group(groups - 1, False)

    pltpu.sync_copy(out_v, out_hbm.at[pl.ds(wid * (BPW * DIM), BPW * DIM)])


_run = pl.kernel(
    _sc_body,
    out_type=jax.ShapeDtypeStruct((BATCH * DIM,), jnp.float32),
    mesh=plsc.VectorSubcoreMesh(core_axis_name="c", subcore_axis_name="s",
                                num_cores=NC, num_subcores=NS),
    scratch_types=[
        pltpu.VMEM((BPW * SEQP,), jnp.int32),
        pltpu.VMEM((NBUF, SEQ, DIM), jnp.float32),
        pltpu.VMEM((BPW * DIM,), jnp.float32),
    ] + [pltpu.SemaphoreType.DMA] * NBUF,
    compiler_params=pltpu.CompilerParams(use_tc_tiling_on_sc=False),
)


def kernel(input_seq_batch, table):
    idx = jnp.pad(input_seq_batch.astype(jnp.int32),
                  ((0, 0), (0, SEQP - SEQ))).reshape(BATCH * SEQP)
    return _run(idx, table).reshape(BATCH, DIM)
